# Initial kernel scaffold; baseline (speedup 1.0000x reference)
import jax
import jax.numpy as jnp
from jax.experimental import pallas as pl


def kernel(x_flight, x_airport, edge_index_fa, edge_index_af, enc_flight_W, enc_flight_b, enc_airport_W, enc_airport_b, conv0_fa_lW, conv0_fa_lb, conv0_fa_rW, conv0_af_lW, conv0_af_lb, conv0_af_rW, conv1_fa_lW, conv1_fa_lb, conv1_fa_rW, conv1_af_lW, conv1_af_lb, conv1_af_rW, readout_W, readout_b):
    raise NotImplementedError("write your pallas kernel here")


# speedup vs baseline: 2.5965x; 2.5965x over previous
import jax
import jax.numpy as jnp
from jax import lax
from jax.experimental import pallas as pl
from jax.experimental.pallas import tpu as pltpu
from jax.experimental.pallas import tpu_sc as plsc

N = 10000
H = 128
E = 160000
NC, NS = 2, 16
NW = NC * NS
CHUNK = 128
NCHUNK = 40
EPW = NCHUNK * CHUNK
E_PAD = NW * EPW
N_PAD = 10240
RPT = N_PAD // NS
SCRAP = N
BR = 1000
G = N // BR

f32 = jnp.float32
i32 = jnp.int32

_sc_mesh = plsc.VectorSubcoreMesh(core_axis_name="c", subcore_axis_name="s",
                                  num_cores=NC, num_subcores=NS)


def _agg_body(table, src, dst, zeros, out, sidx, didx, rows, acc, sem):
    cid = lax.axis_index("c")
    sid = lax.axis_index("s")
    wid = cid * NS + sid
    r0 = sid * RPT
    pltpu.sync_copy(zeros, rows)
    for j in range(RPT // CHUNK):
        pltpu.sync_copy(rows, acc.at[pl.ds(r0 + j * CHUNK, CHUNK)])
    plsc.subcore_barrier()

    def step(i, carry):
        base = wid * EPW + i * CHUNK
        pltpu.sync_copy(src.at[pl.ds(base, CHUNK)], sidx)
        pltpu.sync_copy(dst.at[pl.ds(base, CHUNK)], didx)
        pltpu.async_copy(table.at[sidx], rows, sem).wait()
        pltpu.sync_copy(rows, acc.at[didx], add=True)
        return carry

    lax.fori_loop(0, NCHUNK, step, 0)
    plsc.subcore_barrier()
    for j in range(RPT // CHUNK):
        pltpu.sync_copy(acc.at[pl.ds(r0 + j * CHUNK, CHUNK)], rows)
        pltpu.sync_copy(rows, out.at[cid, pl.ds(r0 + j * CHUNK, CHUNK)])


_agg = pl.kernel(
    _agg_body,
    out_type=jax.ShapeDtypeStruct((NC, N_PAD, H), f32),
    mesh=_sc_mesh,
    scratch_types=[
        pltpu.VMEM((CHUNK,), i32),
        pltpu.VMEM((CHUNK,), i32),
        pltpu.VMEM((CHUNK, H), f32),
        pltpu.VMEM_SHARED((N_PAD, H), f32),
        pltpu.SemaphoreType.DMA,
    ],
    name="sc_segment_sum",
)


def _cnt_body(dst, zeros, ones, out, didx, rows, cacc):
    cid = lax.axis_index("c")
    sid = lax.axis_index("s")
    wid = cid * NS + sid
    r0 = sid * RPT
    pltpu.sync_copy(zeros, rows)
    for j in range(RPT // CHUNK):
        pltpu.sync_copy(rows, cacc.at[pl.ds(r0 + j * CHUNK, CHUNK)])
    pltpu.sync_copy(ones, rows)
    plsc.subcore_barrier()

    def step(i, carry):
        base = wid * EPW + i * CHUNK
        pltpu.sync_copy(dst.at[pl.ds(base, CHUNK)], didx)
        pltpu.sync_copy(rows, cacc.at[didx], add=True)
        return carry

    lax.fori_loop(0, NCHUNK, step, 0)
    plsc.subcore_barrier()
    for j in range(RPT // CHUNK):
        pltpu.sync_copy(cacc.at[pl.ds(r0 + j * CHUNK, CHUNK)], rows)
        pltpu.sync_copy(rows, out.at[cid, pl.ds(r0 + j * CHUNK, CHUNK)])


_cnt = pl.kernel(
    _cnt_body,
    out_type=jax.ShapeDtypeStruct((NC, N_PAD, H), f32),
    mesh=_sc_mesh,
    scratch_types=[
        pltpu.VMEM((CHUNK,), i32),
        pltpu.VMEM((CHUNK, H), f32),
        pltpu.VMEM_SHARED((N_PAD, H), f32),
    ],
    name="sc_degree_count",
)


def _mm(x, w):
    return lax.dot_general(x, w, (((1,), (1,)), ((), ())),
                           preferred_element_type=f32)


def _enc_body(xf, xa, wf, bf, wa, ba, l0fa, r0fa, lb0fa, l0af, r0af, lb0af,
              mf_o, ma_o, ra_o, rf_o):
    hf = jnp.maximum(_mm(xf[...], wf[...]) + bf[...], 0.0)
    ha = jnp.maximum(_mm(xa[...], wa[...]) + ba[...], 0.0)
    mf_o[...] = _mm(hf, l0fa[...])
    ma_o[...] = _mm(ha, l0af[...])
    ra_o[...] = _mm(ha, r0fa[...]) + lb0fa[...]
    rf_o[...] = _mm(hf, r0af[...]) + lb0af[...]


def _mean(agg_ref, cnt_ref, r_ref):
    c = cnt_ref[...]
    cnt = jnp.maximum(c[0, :, 0:1] + c[1, :, 0:1], 1.0)
    a = agg_ref[...]
    return jnp.maximum((a[0] + a[1]) / cnt + r_ref[...], 0.0)


def _mid_body(aggA, cntA, ra0, aggF, cntF, rf0, l1af, r1af, lb1af,
              ma1_o, rf1_o):
    ha1 = _mean(aggA, cntA, ra0)
    ma1_o[...] = _mm(ha1, l1af[...])
    hf1 = _mean(aggF, cntF, rf0)
    rf1_o[...] = _mm(hf1, r1af[...]) + lb1af[...]


def _out_body(aggF, cntF, rf1, ro_w, ro_b, y_o):
    hf2 = _mean(aggF, cntF, rf1)
    y_o[...] = _mm(hf2, ro_w[...]) + ro_b[0, 0]


def _row_spec(cols):
    return pl.BlockSpec((BR, cols), lambda i: (i, 0))


def _full_spec(shape):
    nd = len(shape)
    return pl.BlockSpec(shape, lambda i, _nd=nd: (0,) * _nd)


def _part_spec(cols):
    return pl.BlockSpec((NC, BR, cols), lambda i: (0, i, 0))


_enc_call = pl.pallas_call(
    _enc_body,
    grid=(G,),
    in_specs=[
        _row_spec(H), _row_spec(H),
        _full_spec((H, H)), _full_spec((1, H)),
        _full_spec((H, H)), _full_spec((1, H)),
        _full_spec((H, H)), _full_spec((H, H)), _full_spec((1, H)),
        _full_spec((H, H)), _full_spec((H, H)), _full_spec((1, H)),
    ],
    out_specs=[_row_spec(H)] * 4,
    out_shape=[jax.ShapeDtypeStruct((N, H), f32)] * 4,
)

_mid_call = pl.pallas_call(
    _mid_body,
    grid=(G,),
    in_specs=[
        _part_spec(H), _part_spec(H), _row_spec(H),
        _part_spec(H), _part_spec(H), _row_spec(H),
        _full_spec((H, H)), _full_spec((H, H)), _full_spec((1, H)),
    ],
    out_specs=[_row_spec(H)] * 2,
    out_shape=[jax.ShapeDtypeStruct((N, H), f32)] * 2,
)

_out_call = pl.pallas_call(
    _out_body,
    grid=(G,),
    in_specs=[
        _part_spec(H), _part_spec(H), _row_spec(H),
        _full_spec((H, H)), _full_spec((1, 1)),
    ],
    out_specs=_row_spec(H),
    out_shape=jax.ShapeDtypeStruct((N, H), f32),
)


def kernel(x_flight, x_airport, edge_index_fa, edge_index_af,
           enc_flight_W, enc_flight_b, enc_airport_W, enc_airport_b,
           conv0_fa_lW, conv0_fa_lb, conv0_fa_rW,
           conv0_af_lW, conv0_af_lb, conv0_af_rW,
           conv1_fa_lW, conv1_fa_lb, conv1_fa_rW,
           conv1_af_lW, conv1_af_lb, conv1_af_rW,
           readout_W, readout_b):
    pad = E_PAD - E
    src_fa = jnp.concatenate([edge_index_fa[0], jnp.zeros((pad,), i32)])
    dst_fa = jnp.concatenate([edge_index_fa[1], jnp.full((pad,), SCRAP, i32)])
    src_af = jnp.concatenate([edge_index_af[0], jnp.zeros((pad,), i32)])
    dst_af = jnp.concatenate([edge_index_af[1], jnp.full((pad,), SCRAP, i32)])
    zeros = jnp.zeros((CHUNK, H), f32)
    ones = jnp.ones((CHUNK, H), f32)

    cntA = _cnt(dst_fa, zeros, ones)
    cntF = _cnt(dst_af, zeros, ones)

    mf0, ma0, ra0, rf0 = _enc_call(
        x_flight, x_airport,
        enc_flight_W, enc_flight_b.reshape(1, H),
        enc_airport_W, enc_airport_b.reshape(1, H),
        conv0_fa_lW, conv0_fa_rW, conv0_fa_lb.reshape(1, H),
        conv0_af_lW, conv0_af_rW, conv0_af_lb.reshape(1, H),
    )

    aggA = _agg(mf0, src_fa, dst_fa, zeros)
    aggF = _agg(ma0, src_af, dst_af, zeros)

    ma1, rf1 = _mid_call(
        aggA, cntA, ra0, aggF, cntF, rf0,
        conv1_af_lW, conv1_af_rW, conv1_af_lb.reshape(1, H),
    )

    aggF1 = _agg(ma1, src_af, dst_af, zeros)

    ro_w = jnp.zeros((H, H), f32).at[0:1, :].set(readout_W)
    y = _out_call(aggF1, cntF, rf1, ro_w, readout_b.reshape(1, 1))
    return y[:, 0:1]


# speedup vs baseline: 3.1284x; 1.2049x over previous
import jax
import jax.numpy as jnp
from jax import lax
from jax.experimental import pallas as pl
from jax.experimental.pallas import tpu as pltpu
from jax.experimental.pallas import tpu_sc as plsc

N = 10000
H = 128
E = 160000
NC, NS = 2, 16
NW = NC * NS
CHUNK = 128
NCHUNK = 40
EPW = NCHUNK * CHUNK
E_PAD = NW * EPW
N_PAD = 10240
RPT = N_PAD // NS
SCRAP = N
BR = 1000
G = N // BR

f32 = jnp.float32
i32 = jnp.int32

_sc_mesh = plsc.VectorSubcoreMesh(core_axis_name="c", subcore_axis_name="s",
                                  num_cores=NC, num_subcores=NS)


def _agg_body(table, src, dst, zeros, out, sidx, didx, rows, acc, semA, semB):
    cid = lax.axis_index("c")
    sid = lax.axis_index("s")
    wid = cid * NS + sid
    r0 = sid * RPT
    e0 = wid * EPW
    sems = (semA, semB)

    pltpu.sync_copy(zeros, rows.at[0])
    for j in range(RPT // CHUNK):
        pltpu.sync_copy(rows.at[0], acc.at[pl.ds(r0 + j * CHUNK, CHUNK)])
    plsc.subcore_barrier()

    def load_idx(slot, base):
        pltpu.sync_copy(src.at[pl.ds(base, CHUNK)], sidx.at[slot])
        pltpu.sync_copy(dst.at[pl.ds(base, CHUNK)], didx.at[slot])

    def start_gather(slot):
        pltpu.async_copy(table.at[sidx.at[slot]], rows.at[slot], sems[slot])

    def wait_gather(slot):
        pltpu.make_async_copy(table.at[pl.ds(0, CHUNK)], rows.at[slot],
                              sems[slot]).wait()

    def scatter(slot):
        pltpu.sync_copy(rows.at[slot], acc.at[didx.at[slot]], add=True)

    load_idx(0, e0)
    start_gather(0)
    load_idx(1, e0 + CHUNK)
    start_gather(1)

    def step(g, carry):
        base = e0 + 2 * g * CHUNK
        for slot in range(2):
            wait_gather(slot)
            scatter(slot)
            load_idx(slot, base + (slot + 2) * CHUNK)
            start_gather(slot)
        return carry

    lax.fori_loop(0, NCHUNK // 2 - 1, step, 0)
    for slot in range(2):
        wait_gather(slot)
        scatter(slot)

    plsc.subcore_barrier()
    for j in range(RPT // CHUNK):
        pltpu.sync_copy(acc.at[pl.ds(r0 + j * CHUNK, CHUNK)], rows.at[0])
        pltpu.sync_copy(rows.at[0], out.at[cid, pl.ds(r0 + j * CHUNK, CHUNK)])


_agg = pl.kernel(
    _agg_body,
    out_type=jax.ShapeDtypeStruct((NC, N_PAD, H), f32),
    mesh=_sc_mesh,
    scratch_types=[
        pltpu.VMEM((2, CHUNK), i32),
        pltpu.VMEM((2, CHUNK), i32),
        pltpu.VMEM((2, CHUNK, H), f32),
        pltpu.VMEM_SHARED((N_PAD, H), f32),
        pltpu.SemaphoreType.DMA,
        pltpu.SemaphoreType.DMA,
    ],
    name="sc_segment_sum",
)


def _cnt_body(dst, zeros, ones, out, didx, rows, cacc):
    cid = lax.axis_index("c")
    sid = lax.axis_index("s")
    wid = cid * NS + sid
    r0 = sid * RPT
    pltpu.sync_copy(zeros, rows)
    for j in range(RPT // CHUNK):
        pltpu.sync_copy(rows, cacc.at[pl.ds(r0 + j * CHUNK, CHUNK)])
    pltpu.sync_copy(ones, rows)
    plsc.subcore_barrier()

    def step(i, carry):
        base = wid * EPW + i * CHUNK
        pltpu.sync_copy(dst.at[pl.ds(base, CHUNK)], didx)
        pltpu.sync_copy(rows, cacc.at[didx], add=True)
        return carry

    lax.fori_loop(0, NCHUNK, step, 0)
    plsc.subcore_barrier()
    for j in range(RPT // CHUNK):
        pltpu.sync_copy(cacc.at[pl.ds(r0 + j * CHUNK, CHUNK)], rows)
        pltpu.sync_copy(rows, out.at[cid, pl.ds(r0 + j * CHUNK, CHUNK)])


_cnt = pl.kernel(
    _cnt_body,
    out_type=jax.ShapeDtypeStruct((NC, N_PAD, H), f32),
    mesh=_sc_mesh,
    scratch_types=[
        pltpu.VMEM((CHUNK,), i32),
        pltpu.VMEM((CHUNK, H), f32),
        pltpu.VMEM_SHARED((N_PAD, H), f32),
    ],
    name="sc_degree_count",
)


def _mm(x, w):
    return lax.dot_general(x, w, (((1,), (1,)), ((), ())),
                           preferred_element_type=f32)


def _enc_body(xf, xa, wf, bf, wa, ba, l0fa, r0fa, lb0fa, l0af, r0af, lb0af,
              mf_o, ma_o, ra_o, rf_o):
    hf = jnp.maximum(_mm(xf[...], wf[...]) + bf[...], 0.0)
    ha = jnp.maximum(_mm(xa[...], wa[...]) + ba[...], 0.0)
    mf_o[...] = _mm(hf, l0fa[...])
    ma_o[...] = _mm(ha, l0af[...])
    ra_o[...] = _mm(ha, r0fa[...]) + lb0fa[...]
    rf_o[...] = _mm(hf, r0af[...]) + lb0af[...]


def _mean(agg_ref, cnt_ref, r_ref):
    c = cnt_ref[...]
    cnt = jnp.maximum(c[0, :, 0:1] + c[1, :, 0:1], 1.0)
    a = agg_ref[...]
    return jnp.maximum((a[0] + a[1]) / cnt + r_ref[...], 0.0)


def _mid_body(aggA, cntA, ra0, aggF, cntF, rf0, l1af, r1af, lb1af,
              ma1_o, rf1_o):
    ha1 = _mean(aggA, cntA, ra0)
    ma1_o[...] = _mm(ha1, l1af[...])
    hf1 = _mean(aggF, cntF, rf0)
    rf1_o[...] = _mm(hf1, r1af[...]) + lb1af[...]


def _out_body(aggF, cntF, rf1, ro_w, ro_b, y_o):
    hf2 = _mean(aggF, cntF, rf1)
    y_o[...] = _mm(hf2, ro_w[...]) + ro_b[0, 0]


def _row_spec(cols):
    return pl.BlockSpec((BR, cols), lambda i: (i, 0))


def _full_spec(shape):
    nd = len(shape)
    return pl.BlockSpec(shape, lambda i, _nd=nd: (0,) * _nd)


def _part_spec(cols):
    return pl.BlockSpec((NC, BR, cols), lambda i: (0, i, 0))


_enc_call = pl.pallas_call(
    _enc_body,
    grid=(G,),
    in_specs=[
        _row_spec(H), _row_spec(H),
        _full_spec((H, H)), _full_spec((1, H)),
        _full_spec((H, H)), _full_spec((1, H)),
        _full_spec((H, H)), _full_spec((H, H)), _full_spec((1, H)),
        _full_spec((H, H)), _full_spec((H, H)), _full_spec((1, H)),
    ],
    out_specs=[_row_spec(H)] * 4,
    out_shape=[jax.ShapeDtypeStruct((N, H), f32)] * 4,
)

_mid_call = pl.pallas_call(
    _mid_body,
    grid=(G,),
    in_specs=[
        _part_spec(H), _part_spec(H), _row_spec(H),
        _part_spec(H), _part_spec(H), _row_spec(H),
        _full_spec((H, H)), _full_spec((H, H)), _full_spec((1, H)),
    ],
    out_specs=[_row_spec(H)] * 2,
    out_shape=[jax.ShapeDtypeStruct((N, H), f32)] * 2,
)

_out_call = pl.pallas_call(
    _out_body,
    grid=(G,),
    in_specs=[
        _part_spec(H), _part_spec(H), _row_spec(H),
        _full_spec((H, H)), _full_spec((1, 1)),
    ],
    out_specs=_row_spec(H),
    out_shape=jax.ShapeDtypeStruct((N, H), f32),
)


def kernel(x_flight, x_airport, edge_index_fa, edge_index_af,
           enc_flight_W, enc_flight_b, enc_airport_W, enc_airport_b,
           conv0_fa_lW, conv0_fa_lb, conv0_fa_rW,
           conv0_af_lW, conv0_af_lb, conv0_af_rW,
           conv1_fa_lW, conv1_fa_lb, conv1_fa_rW,
           conv1_af_lW, conv1_af_lb, conv1_af_rW,
           readout_W, readout_b):
    pad = E_PAD - E
    src_fa = jnp.concatenate([edge_index_fa[0], jnp.zeros((pad,), i32)])
    dst_fa = jnp.concatenate([edge_index_fa[1], jnp.full((pad,), SCRAP, i32)])
    src_af = jnp.concatenate([edge_index_af[0], jnp.zeros((pad,), i32)])
    dst_af = jnp.concatenate([edge_index_af[1], jnp.full((pad,), SCRAP, i32)])
    zeros = jnp.zeros((CHUNK, H), f32)
    ones = jnp.ones((CHUNK, H), f32)

    cntA = _cnt(dst_fa, zeros, ones)
    cntF = _cnt(dst_af, zeros, ones)

    mf0, ma0, ra0, rf0 = _enc_call(
        x_flight, x_airport,
        enc_flight_W, enc_flight_b.reshape(1, H),
        enc_airport_W, enc_airport_b.reshape(1, H),
        conv0_fa_lW, conv0_fa_rW, conv0_fa_lb.reshape(1, H),
        conv0_af_lW, conv0_af_rW, conv0_af_lb.reshape(1, H),
    )

    aggA = _agg(mf0, src_fa, dst_fa, zeros)
    aggF = _agg(ma0, src_af, dst_af, zeros)

    ma1, rf1 = _mid_call(
        aggA, cntA, ra0, aggF, cntF, rf0,
        conv1_af_lW, conv1_af_rW, conv1_af_lb.reshape(1, H),
    )

    aggF1 = _agg(ma1, src_af, dst_af, zeros)

    ro_w = jnp.zeros((H, H), f32).at[0:1, :].set(readout_W)
    y = _out_call(aggF1, cntF, rf1, ro_w, readout_b.reshape(1, 1))
    return y[:, 0:1]
